# baseline (speedup 1.0000x reference)
import functools

import jax
import jax.numpy as jnp
from jax import lax
from jax.experimental import pallas as pl
from jax.experimental.pallas import tpu as pltpu
from jax.experimental.pallas import tpu_sc as plsc

N_NODES = 10000
N_EDGES = 320000
DIM = 128
HID = 128
ACT = 8
GRP = 64

NC = 2
NS = 16
NW = NC * NS
CH = 128
EPT = N_EDGES // NW
NFC = EPT // CH
TAIL = EPT - NFC * CH
NPAD = 10112
RPT = NPAD // NS
JUNK = N_NODES

_MESH = plsc.VectorSubcoreMesh(core_axis_name="c", subcore_axis_name="s")


def _deg_body(dst_hbm, out_hbm, dv, cnt, sem):
    ci = lax.axis_index("c")
    si = lax.axis_index("s")
    wid = ci * NS + si

    def z(i, carry):
        cnt[pl.ds(i * 16, 16)] = jnp.zeros((16,), jnp.float32)
        return carry

    lax.fori_loop(0, NPAD // 16, z, 0)
    pltpu.sync_copy(dst_hbm.at[pl.ds(wid * EPT, EPT)], dv)

    ones = jnp.ones((16,), jnp.float32)

    def body(i, carry):
        plsc.addupdate_scatter(cnt, [dv[pl.ds(i * 16, 16)]], ones)
        return carry

    lax.fori_loop(0, EPT // 16, body, 0)
    pltpu.sync_copy(cnt, out_hbm.at[wid])


_deg_call = functools.partial(
    pl.kernel,
    out_type=jax.ShapeDtypeStruct((NW, NPAD), jnp.float32),
    mesh=_MESH,
    compiler_params=pltpu.CompilerParams(needs_layout_passes=False),
    scratch_types=[
        pltpu.VMEM((EPT,), jnp.int32),
        pltpu.VMEM((NPAD,), jnp.float32),
        pltpu.SemaphoreType.DMA,
    ],
)(_deg_body)


def _edge_body(h_hbm, src_hbm, dst_hbm, zeros_hbm, out_hbm,
               sidx, dstv, rows, acc, gsem, ssem, dsem, isem):
    ci = lax.axis_index("c")
    si = lax.axis_index("s")
    wid = ci * NS + si
    base = wid * EPT
    row0 = si * RPT
    pltpu.sync_copy(zeros_hbm.at[pl.ds(row0, RPT)], acc.at[pl.ds(row0, RPT)])
    pltpu.sync_copy(dst_hbm.at[pl.ds(base, CH)], dstv.at[0])
    pltpu.sync_copy(src_hbm.at[pl.ds(base, CH)], sidx.at[0])
    pltpu.async_copy(src_hbm.at[pl.ds(base + CH, CH)], sidx.at[1], isem)
    pltpu.async_copy(h_hbm.at[sidx.at[0]], rows.at[0], gsem.at[0])
    plsc.subcore_barrier()

    def body(c, carry):
        nxt = c + 1

        @pl.when(nxt < NFC)
        def _():
            pltpu.make_async_copy(
                src_hbm.at[pl.ds(base + nxt * CH, CH)], sidx.at[nxt % 2],
                isem).wait()

            @pl.when(c >= 1)
            def _():
                pltpu.make_async_copy(
                    rows.at[nxt % 2], acc.at[dstv.at[nxt % 2]],
                    ssem.at[nxt % 2]).wait()

            pltpu.async_copy(
                h_hbm.at[sidx.at[nxt % 2]], rows.at[nxt % 2],
                gsem.at[nxt % 2])
            pltpu.async_copy(
                dst_hbm.at[pl.ds(base + nxt * CH, CH)], dstv.at[nxt % 2],
                dsem.at[nxt % 2])

        pltpu.make_async_copy(
            h_hbm.at[sidx.at[c % 2]], rows.at[c % 2], gsem.at[c % 2]).wait()

        @pl.when(c + 2 < NFC)
        def _():
            pltpu.async_copy(
                src_hbm.at[pl.ds(base + (c + 2) * CH, CH)], sidx.at[c % 2],
                isem)

        @pl.when(c >= 1)
        def _():
            pltpu.make_async_copy(
                dst_hbm.at[pl.ds(base + c * CH, CH)], dstv.at[c % 2],
                dsem.at[c % 2]).wait()

        pltpu.async_copy(rows.at[c % 2], acc.at[dstv.at[c % 2]],
                         ssem.at[c % 2], add=True)
        return carry

    lax.fori_loop(0, NFC, body, 0)
    pltpu.make_async_copy(
        rows.at[(NFC - 1) % 2], acc.at[dstv.at[(NFC - 1) % 2]],
        ssem.at[(NFC - 1) % 2]).wait()
    pltpu.make_async_copy(
        rows.at[(NFC - 2) % 2], acc.at[dstv.at[(NFC - 2) % 2]],
        ssem.at[(NFC - 2) % 2]).wait()
    tb = base + NFC * CH
    pltpu.sync_copy(src_hbm.at[pl.ds(tb, TAIL)], sidx.at[0, pl.ds(0, TAIL)])
    pltpu.sync_copy(dst_hbm.at[pl.ds(tb, TAIL)], dstv.at[0, pl.ds(0, TAIL)])
    for k in range(TAIL // 16, CH // 16):
        dstv[0, pl.ds(k * 16, 16)] = jnp.full((16,), JUNK + k, jnp.int32)
    pltpu.async_copy(h_hbm.at[sidx.at[0, pl.ds(0, TAIL)]],
                     rows.at[0, pl.ds(0, TAIL)], gsem.at[0])
    pltpu.make_async_copy(h_hbm.at[sidx.at[0, pl.ds(0, TAIL)]],
                          rows.at[0, pl.ds(0, TAIL)], gsem.at[0]).wait()
    pltpu.sync_copy(rows.at[0], acc.at[dstv.at[0]], add=True)
    plsc.subcore_barrier()
    pltpu.sync_copy(acc.at[pl.ds(row0, RPT)], out_hbm.at[ci, pl.ds(row0, RPT)])


_edge_call = functools.partial(
    pl.kernel,
    out_type=jax.ShapeDtypeStruct((NC, NPAD, HID), jnp.float32),
    mesh=_MESH,
    scratch_types=[
        pltpu.VMEM((2, CH), jnp.int32),
        pltpu.VMEM((2, CH), jnp.int32),
        pltpu.VMEM((2, CH, HID), jnp.float32),
        pltpu.VMEM_SHARED((NPAD, HID), jnp.float32),
        pltpu.SemaphoreType.DMA((2,)),
        pltpu.SemaphoreType.DMA((2,)),
        pltpu.SemaphoreType.DMA((2,)),
        pltpu.SemaphoreType.DMA,
    ],
)(_edge_body)


def _mm_body(x_ref, w_ref, h_ref):
    h_ref[...] = jnp.dot(x_ref[...], w_ref[...],
                         preferred_element_type=jnp.float32)


def _mm_call(x, w):
    return pl.pallas_call(
        _mm_body,
        out_shape=jax.ShapeDtypeStruct((N_NODES, HID), jnp.float32),
    )(x, w)


def _prep_body(h_ref, degp_ref, dinv_ref, hs_ref):
    deg = 1.0 + jnp.sum(degp_ref[...][:, : N_NODES], axis=0)
    dinv = 1.0 / jnp.sqrt(deg)
    dinv_b = jnp.broadcast_to(dinv[:, None], (N_NODES, HID))
    dinv_ref[...] = dinv_b[:, :16]
    hs_ref[...] = h_ref[...] * dinv_b


def _prep_call(h, degp):
    return pl.pallas_call(
        _prep_body,
        out_shape=[
            jax.ShapeDtypeStruct((N_NODES, 16), jnp.float32),
            jax.ShapeDtypeStruct((N_NODES, HID), jnp.float32),
        ],
    )(h, degp)


def _mid_body(scat_ref, hs_ref, dinv_ref, b_ref, w_ref, out_ref):
    s = scat_ref[0, : N_NODES, :] + scat_ref[1, : N_NODES, :] + hs_ref[...]
    dinv_b = jnp.broadcast_to(dinv_ref[...][:, :1], (N_NODES, HID))
    h = jnp.maximum(dinv_b * s + b_ref[...], 0.0)
    out_ref[...] = jnp.dot(h, w_ref[...],
                           preferred_element_type=jnp.float32) * dinv_b


def _mid_call(scat, hs, dinv, b, w):
    return pl.pallas_call(
        _mid_body,
        out_shape=jax.ShapeDtypeStruct((N_NODES, HID), jnp.float32),
    )(scat, hs, dinv, b, w)


def _post_body(scat_ref, hs_ref, dinv_ref, b_ref, wa_ref, ba_ref, ls_ref,
               wc1_ref, bc1_ref, wc2_ref, bc2_ref, batch_ref,
               mean_ref, std_ref, value_ref):
    s = scat_ref[0, : N_NODES, :] + scat_ref[1, : N_NODES, :] + hs_ref[...]
    dinv_b = jnp.broadcast_to(dinv_ref[...][:, :1], (N_NODES, HID))
    h2 = jnp.maximum(dinv_b * s + b_ref[...], 0.0)
    mean_ref[...] = jnp.tanh(
        jnp.dot(h2, wa_ref[...], preferred_element_type=jnp.float32)
        + ba_ref[...])
    std_ref[...] = jnp.exp(jnp.broadcast_to(ls_ref[...], (N_NODES, ACT)))
    seg = lax.broadcasted_iota(jnp.int32, (1, GRP), 1)
    onehot = (batch_ref[...] == seg).astype(jnp.float32)
    sums = lax.dot_general(onehot, h2, (((0,), (0,)), ((), ())),
                           preferred_element_type=jnp.float32)
    cnts = lax.dot_general(onehot, jnp.ones((N_NODES, 1), jnp.float32),
                           (((0,), (0,)), ((), ())),
                           preferred_element_type=jnp.float32)
    gx = sums / jnp.maximum(cnts, 1.0)
    hid = jnp.maximum(
        jnp.dot(gx, wc1_ref[...], preferred_element_type=jnp.float32)
        + bc1_ref[...], 0.0)
    value_ref[...] = (
        jnp.dot(hid, wc2_ref[...], preferred_element_type=jnp.float32)
        + bc2_ref[...])


def _post_call(scat, hs, dinv, b, wa, ba, ls, wc1, bc1, wc2, bc2, batch2d):
    return pl.pallas_call(
        _post_body,
        out_shape=[
            jax.ShapeDtypeStruct((N_NODES, ACT), jnp.float32),
            jax.ShapeDtypeStruct((N_NODES, ACT), jnp.float32),
            jax.ShapeDtypeStruct((GRP, 1), jnp.float32),
        ],
    )(scat, hs, dinv, b, wa, ba, ls, wc1, bc1, wc2, bc2, batch2d)


def kernel(x, edge_index, batch, W1, b1, W2, b2, Wa, ba, log_std,
           Wc1, bc1, Wc2, bc2):
    zeros_h = jnp.zeros((NPAD, HID), jnp.float32)
    src = edge_index[0]
    dst = edge_index[1]

    h0 = _mm_call(x, W1)
    degp = _deg_call(dst)
    dinv, hs1 = _prep_call(h0, degp)
    scat1 = _edge_call(hs1, src, dst, zeros_h)
    hs2 = _mid_call(scat1, hs1, dinv, b1.reshape(1, HID), W2)
    scat2 = _edge_call(hs2, src, dst, zeros_h)
    mean, std, value = _post_call(
        scat2, hs2, dinv, b2.reshape(1, HID), Wa, ba.reshape(1, ACT),
        log_std, Wc1, bc1.reshape(1, GRP), Wc2, bc2.reshape(1, 1),
        batch.reshape(N_NODES, 1))
    return (mean, std, value)


# speedup vs baseline: 1.0481x; 1.0481x over previous
import functools

import jax
import jax.numpy as jnp
from jax import lax
from jax.experimental import pallas as pl
from jax.experimental.pallas import tpu as pltpu
from jax.experimental.pallas import tpu_sc as plsc

N_NODES = 10000
N_EDGES = 320000
DIM = 128
HID = 128
ACT = 8
GRP = 64

NC = 2
NS = 16
NW = NC * NS
CH = 128
EPT = N_EDGES // NW
NFC = EPT // CH
TAIL = EPT - NFC * CH
NPAD = 10112
RPT = NPAD // NS
JUNK = N_NODES

_MESH = plsc.VectorSubcoreMesh(core_axis_name="c", subcore_axis_name="s")


def _deg_body(dst_hbm, out_hbm, dv, cnt, sem):
    ci = lax.axis_index("c")
    si = lax.axis_index("s")
    wid = ci * NS + si

    def z(i, carry):
        cnt[pl.ds(i * 16, 16)] = jnp.zeros((16,), jnp.float32)
        return carry

    lax.fori_loop(0, NPAD // 16, z, 0)
    pltpu.sync_copy(dst_hbm.at[pl.ds(wid * EPT, EPT)], dv)

    ones = jnp.ones((16,), jnp.float32)

    def body(i, carry):
        plsc.addupdate_scatter(cnt, [dv[pl.ds(i * 16, 16)]], ones)
        return carry

    lax.fori_loop(0, EPT // 16, body, 0)
    pltpu.sync_copy(cnt, out_hbm.at[wid])


_deg_call = functools.partial(
    pl.kernel,
    out_type=jax.ShapeDtypeStruct((NW, NPAD), jnp.float32),
    mesh=_MESH,
    compiler_params=pltpu.CompilerParams(needs_layout_passes=False),
    scratch_types=[
        pltpu.VMEM((EPT,), jnp.int32),
        pltpu.VMEM((NPAD,), jnp.float32),
        pltpu.SemaphoreType.DMA,
    ],
)(_deg_body)


def _edge_body(h_hbm, src_hbm, dst_hbm, zeros_hbm, out_hbm,
               sidx, dstv, rows, acc, gsem, ssem, dsem, isem):
    ci = lax.axis_index("c")
    si = lax.axis_index("s")
    wid = ci * NS + si
    base = wid * EPT
    row0 = si * RPT
    pltpu.sync_copy(zeros_hbm.at[pl.ds(row0, RPT)], acc.at[pl.ds(row0, RPT)])
    pltpu.sync_copy(dst_hbm.at[pl.ds(base, CH)], dstv.at[0])
    pltpu.sync_copy(src_hbm.at[pl.ds(base, CH)], sidx.at[0])
    pltpu.async_copy(src_hbm.at[pl.ds(base + CH, CH)], sidx.at[1], isem)
    pltpu.async_copy(h_hbm.at[sidx.at[0]], rows.at[0], gsem.at[0])
    plsc.subcore_barrier()

    def body(c, carry):
        nxt = c + 1

        @pl.when(nxt < NFC)
        def _():
            pltpu.make_async_copy(
                src_hbm.at[pl.ds(base + nxt * CH, CH)], sidx.at[nxt % 2],
                isem).wait()

            @pl.when(c >= 1)
            def _():
                pltpu.make_async_copy(
                    rows.at[nxt % 2], acc.at[dstv.at[nxt % 2]],
                    ssem.at[nxt % 2]).wait()

            pltpu.async_copy(
                h_hbm.at[sidx.at[nxt % 2]], rows.at[nxt % 2],
                gsem.at[nxt % 2])
            pltpu.async_copy(
                dst_hbm.at[pl.ds(base + nxt * CH, CH)], dstv.at[nxt % 2],
                dsem.at[nxt % 2])

        pltpu.make_async_copy(
            h_hbm.at[sidx.at[c % 2]], rows.at[c % 2], gsem.at[c % 2]).wait()

        @pl.when(c + 2 < NFC)
        def _():
            pltpu.async_copy(
                src_hbm.at[pl.ds(base + (c + 2) * CH, CH)], sidx.at[c % 2],
                isem)

        @pl.when(c >= 1)
        def _():
            pltpu.make_async_copy(
                dst_hbm.at[pl.ds(base + c * CH, CH)], dstv.at[c % 2],
                dsem.at[c % 2]).wait()

        pltpu.async_copy(rows.at[c % 2], acc.at[dstv.at[c % 2]],
                         ssem.at[c % 2], add=True)
        return carry

    lax.fori_loop(0, NFC, body, 0)
    pltpu.make_async_copy(
        rows.at[(NFC - 1) % 2], acc.at[dstv.at[(NFC - 1) % 2]],
        ssem.at[(NFC - 1) % 2]).wait()
    pltpu.make_async_copy(
        rows.at[(NFC - 2) % 2], acc.at[dstv.at[(NFC - 2) % 2]],
        ssem.at[(NFC - 2) % 2]).wait()
    tb = base + NFC * CH
    pltpu.sync_copy(src_hbm.at[pl.ds(tb, TAIL)], sidx.at[0, pl.ds(0, TAIL)])
    pltpu.sync_copy(dst_hbm.at[pl.ds(tb, TAIL)], dstv.at[0, pl.ds(0, TAIL)])
    for k in range(TAIL // 16, CH // 16):
        dstv[0, pl.ds(k * 16, 16)] = jnp.full((16,), JUNK + k, jnp.int32)
    pltpu.async_copy(h_hbm.at[sidx.at[0, pl.ds(0, TAIL)]],
                     rows.at[0, pl.ds(0, TAIL)], gsem.at[0])
    pltpu.make_async_copy(h_hbm.at[sidx.at[0, pl.ds(0, TAIL)]],
                          rows.at[0, pl.ds(0, TAIL)], gsem.at[0]).wait()
    pltpu.sync_copy(rows.at[0], acc.at[dstv.at[0]], add=True)
    plsc.subcore_barrier()
    pltpu.sync_copy(acc.at[pl.ds(row0, RPT)], out_hbm.at[ci, pl.ds(row0, RPT)])


_edge_call = functools.partial(
    pl.kernel,
    out_type=jax.ShapeDtypeStruct((NC, NPAD, HID), jnp.float32),
    mesh=_MESH,
    scratch_types=[
        pltpu.VMEM((2, CH), jnp.int32),
        pltpu.VMEM((2, CH), jnp.int32),
        pltpu.VMEM((2, CH, HID), jnp.float32),
        pltpu.VMEM_SHARED((NPAD, HID), jnp.float32),
        pltpu.SemaphoreType.DMA((2,)),
        pltpu.SemaphoreType.DMA((2,)),
        pltpu.SemaphoreType.DMA((2,)),
        pltpu.SemaphoreType.DMA,
    ],
)(_edge_body)


def _split_body(ei_ref, src_ref, dst_ref):
    src_ref[...] = ei_ref[0, :]
    dst_ref[...] = ei_ref[1, :]


def _split_call(edge_index):
    return pl.pallas_call(
        _split_body,
        out_shape=[
            jax.ShapeDtypeStruct((N_EDGES,), jnp.int32),
            jax.ShapeDtypeStruct((N_EDGES,), jnp.int32),
        ],
    )(edge_index)


def _mm_body(x_ref, w_ref, h_ref):
    h_ref[...] = jnp.dot(x_ref[...], w_ref[...],
                         preferred_element_type=jnp.float32)


def _mm_call(x, w):
    return pl.pallas_call(
        _mm_body,
        out_shape=jax.ShapeDtypeStruct((N_NODES, HID), jnp.float32),
    )(x, w)


def _prep_body(h_ref, degp_ref, dinv_ref, hs_ref):
    deg = 1.0 + jnp.sum(degp_ref[...][:, : N_NODES], axis=0)
    dinv = 1.0 / jnp.sqrt(deg)
    dinv_b = jnp.broadcast_to(dinv[:, None], (N_NODES, HID))
    dinv_ref[...] = dinv_b[:, :16]
    hs_ref[...] = h_ref[...] * dinv_b


def _prep_call(h, degp):
    return pl.pallas_call(
        _prep_body,
        out_shape=[
            jax.ShapeDtypeStruct((N_NODES, 16), jnp.float32),
            jax.ShapeDtypeStruct((N_NODES, HID), jnp.float32),
        ],
    )(h, degp)


def _mid_body(scat_ref, hs_ref, dinv_ref, b_ref, w_ref, out_ref):
    s = scat_ref[0, : N_NODES, :] + scat_ref[1, : N_NODES, :] + hs_ref[...]
    dinv_b = jnp.broadcast_to(dinv_ref[...][:, :1], (N_NODES, HID))
    h = jnp.maximum(dinv_b * s + b_ref[...], 0.0)
    out_ref[...] = jnp.dot(h, w_ref[...],
                           preferred_element_type=jnp.float32) * dinv_b


def _mid_call(scat, hs, dinv, b, w):
    return pl.pallas_call(
        _mid_body,
        out_shape=jax.ShapeDtypeStruct((N_NODES, HID), jnp.float32),
    )(scat, hs, dinv, b, w)


def _post_body(scat_ref, hs_ref, dinv_ref, b_ref, wa_ref, ba_ref, ls_ref,
               wc1_ref, bc1_ref, wc2_ref, bc2_ref, batch_ref,
               mean_ref, std_ref, value_ref):
    s = scat_ref[0, : N_NODES, :] + scat_ref[1, : N_NODES, :] + hs_ref[...]
    dinv_b = jnp.broadcast_to(dinv_ref[...][:, :1], (N_NODES, HID))
    h2 = jnp.maximum(dinv_b * s + b_ref[...], 0.0)
    mean_ref[...] = jnp.tanh(
        jnp.dot(h2, wa_ref[...], preferred_element_type=jnp.float32)
        + ba_ref[...])
    std_ref[...] = jnp.exp(jnp.broadcast_to(ls_ref[...], (N_NODES, ACT)))
    seg = lax.broadcasted_iota(jnp.int32, (1, GRP), 1)
    onehot = (batch_ref[...] == seg).astype(jnp.float32)
    sums = lax.dot_general(onehot, h2, (((0,), (0,)), ((), ())),
                           preferred_element_type=jnp.float32)
    cnts = lax.dot_general(onehot, jnp.ones((N_NODES, 1), jnp.float32),
                           (((0,), (0,)), ((), ())),
                           preferred_element_type=jnp.float32)
    gx = sums / jnp.maximum(cnts, 1.0)
    hid = jnp.maximum(
        jnp.dot(gx, wc1_ref[...], preferred_element_type=jnp.float32)
        + bc1_ref[...], 0.0)
    value_ref[...] = (
        jnp.dot(hid, wc2_ref[...], preferred_element_type=jnp.float32)
        + bc2_ref[...])


def _post_call(scat, hs, dinv, b, wa, ba, ls, wc1, bc1, wc2, bc2, batch2d):
    return pl.pallas_call(
        _post_body,
        out_shape=[
            jax.ShapeDtypeStruct((N_NODES, ACT), jnp.float32),
            jax.ShapeDtypeStruct((N_NODES, ACT), jnp.float32),
            jax.ShapeDtypeStruct((GRP, 1), jnp.float32),
        ],
    )(scat, hs, dinv, b, wa, ba, ls, wc1, bc1, wc2, bc2, batch2d)


def kernel(x, edge_index, batch, W1, b1, W2, b2, Wa, ba, log_std,
           Wc1, bc1, Wc2, bc2):
    zeros_h = jnp.zeros((NPAD, HID), jnp.float32)
    src, dst = _split_call(edge_index)

    h0 = _mm_call(x, W1)
    degp = _deg_call(dst)
    dinv, hs1 = _prep_call(h0, degp)
    scat1 = _edge_call(hs1, src, dst, zeros_h)
    hs2 = _mid_call(scat1, hs1, dinv, b1.reshape(1, HID), W2)
    scat2 = _edge_call(hs2, src, dst, zeros_h)
    mean, std, value = _post_call(
        scat2, hs2, dinv, b2.reshape(1, HID), Wa, ba.reshape(1, ACT),
        log_std, Wc1, bc1.reshape(1, GRP), Wc2, bc2.reshape(1, 1),
        batch.reshape(N_NODES, 1))
    return (mean, std, value)


# speedup vs baseline: 1.0524x; 1.0041x over previous
import functools

import jax
import jax.numpy as jnp
from jax import lax
from jax.experimental import pallas as pl
from jax.experimental.pallas import tpu as pltpu
from jax.experimental.pallas import tpu_sc as plsc

N_NODES = 10000
N_EDGES = 320000
DIM = 128
HID = 128
ACT = 8
GRP = 64

NC = 2
NS = 16
NW = NC * NS
CH = 128
EPT = N_EDGES // NW
NFC = EPT // CH
TAIL = EPT - NFC * CH
NPAD = 10112
RPT = NPAD // NS
JUNK = N_NODES

_MESH = plsc.VectorSubcoreMesh(core_axis_name="c", subcore_axis_name="s")


def _deg_body(dst_hbm, out_hbm, dv, cnt, sem):
    ci = lax.axis_index("c")
    si = lax.axis_index("s")
    wid = ci * NS + si

    def z(i, carry):
        cnt[pl.ds(i * 16, 16)] = jnp.zeros((16,), jnp.float32)
        return carry

    lax.fori_loop(0, NPAD // 16, z, 0)
    pltpu.sync_copy(dst_hbm.at[pl.ds(wid * EPT, EPT)], dv)

    ones = jnp.ones((16,), jnp.float32)

    def body(i, carry):
        plsc.addupdate_scatter(cnt, [dv[pl.ds(i * 16, 16)]], ones)
        return carry

    lax.fori_loop(0, EPT // 16, body, 0)
    pltpu.sync_copy(cnt, out_hbm.at[wid])


_deg_call = functools.partial(
    pl.kernel,
    out_type=jax.ShapeDtypeStruct((NW, NPAD), jnp.float32),
    mesh=_MESH,
    compiler_params=pltpu.CompilerParams(needs_layout_passes=False),
    scratch_types=[
        pltpu.VMEM((EPT,), jnp.int32),
        pltpu.VMEM((NPAD,), jnp.float32),
        pltpu.SemaphoreType.DMA,
    ],
)(_deg_body)


def _edge_body(h_hbm, src_hbm, dst_hbm, zeros_hbm, out_hbm,
               sidx, dstv, rows, acc, gsem, ssem, dsem, isem):
    ci = lax.axis_index("c")
    si = lax.axis_index("s")
    wid = ci * NS + si
    base = wid * EPT
    row0 = si * RPT
    pltpu.sync_copy(zeros_hbm.at[pl.ds(row0, RPT)], acc.at[pl.ds(row0, RPT)])
    pltpu.sync_copy(dst_hbm.at[pl.ds(base, CH)], dstv.at[0])
    pltpu.sync_copy(src_hbm.at[pl.ds(base, CH)], sidx.at[0])
    pltpu.async_copy(src_hbm.at[pl.ds(base + CH, CH)], sidx.at[1], isem)
    pltpu.async_copy(h_hbm.at[sidx.at[0]], rows.at[0], gsem.at[0])
    plsc.subcore_barrier()

    def body(c, carry):
        nxt = c + 1

        @pl.when(nxt < NFC)
        def _():
            pltpu.make_async_copy(
                src_hbm.at[pl.ds(base + nxt * CH, CH)], sidx.at[nxt % 2],
                isem).wait()

            @pl.when(c >= 1)
            def _():
                pltpu.make_async_copy(
                    rows.at[nxt % 2], acc.at[dstv.at[nxt % 2]],
                    ssem.at[nxt % 2]).wait()

            pltpu.async_copy(
                h_hbm.at[sidx.at[nxt % 2]], rows.at[nxt % 2],
                gsem.at[nxt % 2])
            pltpu.async_copy(
                dst_hbm.at[pl.ds(base + nxt * CH, CH)], dstv.at[nxt % 2],
                dsem.at[nxt % 2])

        pltpu.make_async_copy(
            h_hbm.at[sidx.at[c % 2]], rows.at[c % 2], gsem.at[c % 2]).wait()

        @pl.when(c + 2 < NFC)
        def _():
            pltpu.async_copy(
                src_hbm.at[pl.ds(base + (c + 2) * CH, CH)], sidx.at[c % 2],
                isem)

        @pl.when(c >= 1)
        def _():
            pltpu.make_async_copy(
                dst_hbm.at[pl.ds(base + c * CH, CH)], dstv.at[c % 2],
                dsem.at[c % 2]).wait()

        pltpu.async_copy(rows.at[c % 2], acc.at[dstv.at[c % 2]],
                         ssem.at[c % 2], add=True)
        return carry

    lax.fori_loop(0, NFC, body, 0)
    pltpu.make_async_copy(
        rows.at[(NFC - 1) % 2], acc.at[dstv.at[(NFC - 1) % 2]],
        ssem.at[(NFC - 1) % 2]).wait()
    pltpu.make_async_copy(
        rows.at[(NFC - 2) % 2], acc.at[dstv.at[(NFC - 2) % 2]],
        ssem.at[(NFC - 2) % 2]).wait()
    tb = base + NFC * CH
    pltpu.sync_copy(src_hbm.at[pl.ds(tb, TAIL)], sidx.at[0, pl.ds(0, TAIL)])
    pltpu.sync_copy(dst_hbm.at[pl.ds(tb, TAIL)], dstv.at[0, pl.ds(0, TAIL)])
    for k in range(TAIL // 16, CH // 16):
        dstv[0, pl.ds(k * 16, 16)] = jnp.full((16,), JUNK + k, jnp.int32)
    pltpu.async_copy(h_hbm.at[sidx.at[0, pl.ds(0, TAIL)]],
                     rows.at[0, pl.ds(0, TAIL)], gsem.at[0])
    pltpu.make_async_copy(h_hbm.at[sidx.at[0, pl.ds(0, TAIL)]],
                          rows.at[0, pl.ds(0, TAIL)], gsem.at[0]).wait()
    pltpu.sync_copy(rows.at[0], acc.at[dstv.at[0]], add=True)
    plsc.subcore_barrier()
    pltpu.sync_copy(acc.at[pl.ds(row0, RPT)], out_hbm.at[ci, pl.ds(row0, RPT)])


_edge_call = functools.partial(
    pl.kernel,
    out_type=jax.ShapeDtypeStruct((NC, NPAD, HID), jnp.float32),
    mesh=_MESH,
    scratch_types=[
        pltpu.VMEM((2, CH), jnp.int32),
        pltpu.VMEM((2, CH), jnp.int32),
        pltpu.VMEM((2, CH, HID), jnp.float32),
        pltpu.VMEM_SHARED((NPAD, HID), jnp.float32),
        pltpu.SemaphoreType.DMA((2,)),
        pltpu.SemaphoreType.DMA((2,)),
        pltpu.SemaphoreType.DMA((2,)),
        pltpu.SemaphoreType.DMA,
    ],
)(_edge_body)


def _split_body(ei_ref, src_ref, dst_ref, zeros_ref):
    src_ref[...] = ei_ref[0, :]
    dst_ref[...] = ei_ref[1, :]
    zeros_ref[...] = jnp.zeros((NPAD, HID), jnp.float32)


def _split_call(edge_index):
    return pl.pallas_call(
        _split_body,
        out_shape=[
            jax.ShapeDtypeStruct((N_EDGES,), jnp.int32),
            jax.ShapeDtypeStruct((N_EDGES,), jnp.int32),
            jax.ShapeDtypeStruct((NPAD, HID), jnp.float32),
        ],
    )(edge_index)


def _mm_body(x_ref, w_ref, h_ref):
    h_ref[...] = jnp.dot(x_ref[...], w_ref[...],
                         preferred_element_type=jnp.float32)


def _mm_call(x, w):
    return pl.pallas_call(
        _mm_body,
        out_shape=jax.ShapeDtypeStruct((N_NODES, HID), jnp.float32),
    )(x, w)


def _prep_body(h_ref, degp_ref, dinv_ref, hs_ref):
    deg = 1.0 + jnp.sum(degp_ref[...][:, : N_NODES], axis=0)
    dinv = 1.0 / jnp.sqrt(deg)
    dinv_b = jnp.broadcast_to(dinv[:, None], (N_NODES, HID))
    dinv_ref[...] = dinv_b[:, :16]
    hs_ref[...] = h_ref[...] * dinv_b


def _prep_call(h, degp):
    return pl.pallas_call(
        _prep_body,
        out_shape=[
            jax.ShapeDtypeStruct((N_NODES, 16), jnp.float32),
            jax.ShapeDtypeStruct((N_NODES, HID), jnp.float32),
        ],
    )(h, degp)


def _mid_body(scat_ref, hs_ref, dinv_ref, b_ref, w_ref, out_ref):
    s = scat_ref[0, : N_NODES, :] + scat_ref[1, : N_NODES, :] + hs_ref[...]
    dinv_b = jnp.broadcast_to(dinv_ref[...][:, :1], (N_NODES, HID))
    h = jnp.maximum(dinv_b * s + b_ref[...], 0.0)
    out_ref[...] = jnp.dot(h, w_ref[...],
                           preferred_element_type=jnp.float32) * dinv_b


def _mid_call(scat, hs, dinv, b, w):
    return pl.pallas_call(
        _mid_body,
        out_shape=jax.ShapeDtypeStruct((N_NODES, HID), jnp.float32),
    )(scat, hs, dinv, b, w)


def _post_body(scat_ref, hs_ref, dinv_ref, b_ref, wa_ref, ba_ref, ls_ref,
               wc1_ref, bc1_ref, wc2_ref, bc2_ref, batch_ref,
               mean_ref, std_ref, value_ref):
    s = scat_ref[0, : N_NODES, :] + scat_ref[1, : N_NODES, :] + hs_ref[...]
    dinv_b = jnp.broadcast_to(dinv_ref[...][:, :1], (N_NODES, HID))
    h2 = jnp.maximum(dinv_b * s + b_ref[...], 0.0)
    mean_ref[...] = jnp.tanh(
        jnp.dot(h2, wa_ref[...], preferred_element_type=jnp.float32)
        + ba_ref[...])
    std_ref[...] = jnp.exp(jnp.broadcast_to(ls_ref[...], (N_NODES, ACT)))
    seg = lax.broadcasted_iota(jnp.int32, (1, GRP), 1)
    onehot = (batch_ref[...] == seg).astype(jnp.float32)
    sums = lax.dot_general(onehot, h2, (((0,), (0,)), ((), ())),
                           preferred_element_type=jnp.float32)
    cnts = lax.dot_general(onehot, jnp.ones((N_NODES, 1), jnp.float32),
                           (((0,), (0,)), ((), ())),
                           preferred_element_type=jnp.float32)
    gx = sums / jnp.maximum(cnts, 1.0)
    hid = jnp.maximum(
        jnp.dot(gx, wc1_ref[...], preferred_element_type=jnp.float32)
        + bc1_ref[...], 0.0)
    value_ref[...] = (
        jnp.dot(hid, wc2_ref[...], preferred_element_type=jnp.float32)
        + bc2_ref[...])


def _post_call(scat, hs, dinv, b, wa, ba, ls, wc1, bc1, wc2, bc2, batch2d):
    return pl.pallas_call(
        _post_body,
        out_shape=[
            jax.ShapeDtypeStruct((N_NODES, ACT), jnp.float32),
            jax.ShapeDtypeStruct((N_NODES, ACT), jnp.float32),
            jax.ShapeDtypeStruct((GRP, 1), jnp.float32),
        ],
    )(scat, hs, dinv, b, wa, ba, ls, wc1, bc1, wc2, bc2, batch2d)


def kernel(x, edge_index, batch, W1, b1, W2, b2, Wa, ba, log_std,
           Wc1, bc1, Wc2, bc2):
    src, dst, zeros_h = _split_call(edge_index)

    h0 = _mm_call(x, W1)
    degp = _deg_call(dst)
    dinv, hs1 = _prep_call(h0, degp)
    scat1 = _edge_call(hs1, src, dst, zeros_h)
    hs2 = _mid_call(scat1, hs1, dinv, b1.reshape(1, HID), W2)
    scat2 = _edge_call(hs2, src, dst, zeros_h)
    mean, std, value = _post_call(
        scat2, hs2, dinv, b2.reshape(1, HID), Wa, ba.reshape(1, ACT),
        log_std, Wc1, bc1.reshape(1, GRP), Wc2, bc2.reshape(1, 1),
        batch.reshape(N_NODES, 1))
    return (mean, std, value)
